# baseline (speedup 1.0000x reference)
import functools

import jax
import jax.numpy as jnp
from jax import lax
from jax.experimental import pallas as pl
from jax.experimental.pallas import tpu as pltpu
from jax.experimental.pallas import tpu_sc as plsc

_VOCAB = 30522
_HIDDEN = 768
_L = 16
_EPS = 1e-12

_NC, _NS = 2, 16
_NW = _NC * _NS
_SEQ = 128
_BATCH = 1024
_TOK = _BATCH * _SEQ
_TPW = _TOK // _NW
_SPW = _TPW // _SEQ
_C = 64
_NCHUNK = _SEQ // _C
_G = 16
_KPS = _C // _G
_NBUF = 4
_U = 4


def _newton_rsqrt(x):
    i = plsc.bitcast(x, jnp.int32)
    i = 0x5F3759DF - (i >> 1)
    y = plsc.bitcast(i, jnp.float32)
    for _ in range(3):
        y = y * (1.5 - 0.5 * x * y * y)
    return y


_mesh = plsc.VectorSubcoreMesh(core_axis_name="c", subcore_axis_name="s")


@functools.partial(
    pl.kernel,
    mesh=_mesh,
    compiler_params=pltpu.CompilerParams(
        needs_layout_passes=False, use_tc_tiling_on_sc=False),
    out_type=jax.ShapeDtypeStruct((_TOK, _HIDDEN), jnp.float32),
    scratch_types=(
        [pltpu.VMEM((_SPW, _NCHUNK, _C), jnp.int32),
         pltpu.VMEM((_HIDDEN, _C), jnp.float32),
         pltpu.VMEM((_HIDDEN, _L), jnp.float32),
         pltpu.VMEM((1, _HIDDEN), jnp.float32),
         pltpu.SMEM((_HIDDEN,), jnp.float32),
         pltpu.SMEM((_HIDDEN,), jnp.float32)]
        + [pltpu.VMEM((_G, _HIDDEN), jnp.float32) for _ in range(_NBUF)]
        + [pltpu.SemaphoreType.DMA for _ in range(2 * _NBUF)]
    ),
)
def _emb_kernel(word_hbm, idx_hbm, posT_hbm, gam_hbm, bet_hbm,
                out_hbm, idx_l, biasT, temp, gb_v, gam_s, bet_s,
                *bufs_sems):
    bufs = bufs_sems[:_NBUF]
    gsem = bufs_sems[_NBUF:2 * _NBUF]
    ssem = bufs_sems[2 * _NBUF:]
    wid = lax.axis_index("s") * _NC + lax.axis_index("c")
    iota = lax.iota(jnp.int32, _L)

    pltpu.sync_copy(idx_hbm.at[wid], idx_l)
    for src, dst in ((gam_hbm, gam_s), (bet_hbm, bet_s)):
        pltpu.sync_copy(src, gb_v)
        for t in range(_HIDDEN // _L):
            v = gb_v[0, pl.ds(t * _L, _L)]
            for u in range(_L):
                dst[t * _L + u] = v[u]

    def compute_tile(buf, k):
        zero = jnp.zeros((_L,), jnp.float32)
        csl = pl.ds(k * _G, _L)

        def p1_body(t, carry):
            colv = carry[0]
            accs = list(carry[1:])
            for u in range(_U):
                e = _U * t + u
                v = plsc.load_gather(buf, [iota, colv])
                emb = v + biasT[e, csl]
                temp[e, :] = emb
                accs[u] = accs[u] + emb
                accs[_U + u] = accs[_U + u] + emb * emb
                colv = colv + 1
            return (colv, *accs)

        res = lax.fori_loop(
            0, _HIDDEN // _U, p1_body,
            (jnp.zeros((_L,), jnp.int32),) + (zero,) * (2 * _U))
        s1 = res[1] + res[2] + res[3] + res[4]
        s2 = res[5] + res[6] + res[7] + res[8]
        m = s1 * (1.0 / _HIDDEN)
        var = s2 * (1.0 / _HIDDEN) - m * m
        rs = _newton_rsqrt(var + _EPS)

        def p2_body(t, colv):
            for u in range(_U):
                e = _U * t + u
                v = temp[e, :]
                o = (v - m) * rs * gam_s[e] + bet_s[e]
                plsc.store_scatter(buf, [iota, colv], o)
                colv = colv + 1
            return colv

        lax.fori_loop(0, _HIDDEN // _U, p2_body, jnp.zeros((_L,), jnp.int32))

    for c in range(_NCHUNK):
        pltpu.sync_copy(posT_hbm.at[c], biasT)

        for k in range(2):
            pltpu.async_copy(
                word_hbm.at[idx_l.at[0, c, pl.ds(k * _G, _G)]],
                bufs[k], gsem[k])

        def seq_body(s, carry, c=c):
            for k in range(_KPS):
                k2 = (k + 2) % _NBUF
                pltpu.make_async_copy(
                    word_hbm.at[pl.ds(0, _G)], bufs[k], gsem[k]).wait()
                compute_tile(bufs[k], k)
                obase = wid * _TPW + s * _SEQ + c * _C + k * _G
                pltpu.async_copy(
                    bufs[k], out_hbm.at[pl.ds(obase, _G)], ssem[k])
                if k < 2:
                    @pl.when(s >= 1)
                    def _():
                        pltpu.make_async_copy(
                            bufs[k2], out_hbm.at[pl.ds(0, _G)],
                            ssem[k2]).wait()
                    pltpu.async_copy(
                        word_hbm.at[idx_l.at[s, c, pl.ds(k2 * _G, _G)]],
                        bufs[k2], gsem[k2])
                else:
                    pltpu.make_async_copy(
                        bufs[k2], out_hbm.at[pl.ds(0, _G)],
                        ssem[k2]).wait()

                    @pl.when(s < _SPW - 1)
                    def _():
                        pltpu.async_copy(
                            word_hbm.at[idx_l.at[s + 1, c,
                                                 pl.ds(k2 * _G, _G)]],
                            bufs[k2], gsem[k2])
            return carry

        lax.fori_loop(0, _SPW, seq_body, 0)

        for k in (2, 3):
            pltpu.make_async_copy(
                bufs[k], out_hbm.at[pl.ds(0, _G)], ssem[k]).wait()


def kernel(input_tokens, word_table, pos_table, type_table, ln_gamma, ln_beta):
    idx = input_tokens.astype(jnp.int32).reshape(_NW, _SPW, _NCHUNK, _C)
    bias = pos_table[:_SEQ] + type_table[0][None, :]
    posT = bias.reshape(_NCHUNK, _C, _HIDDEN).transpose(0, 2, 1)
    out = _emb_kernel(word_table, idx, posT,
                      ln_gamma.reshape(1, _HIDDEN),
                      ln_beta.reshape(1, _HIDDEN))
    return out.reshape(_BATCH, _SEQ, _HIDDEN)


# speedup vs baseline: 4.8743x; 4.8743x over previous
import functools

import jax
import jax.numpy as jnp
from jax import lax
from jax.experimental import pallas as pl
from jax.experimental.pallas import tpu as pltpu
from jax.experimental.pallas import tpu_sc as plsc

_VOCAB = 30522
_HIDDEN = 768
_L = 16
_NV = _HIDDEN // _L
_EPS = 1e-12

_NC, _NS = 2, 16
_NW = _NC * _NS
_SEQ = 128
_BATCH = 1024
_TOK = _BATCH * _SEQ
_TPW = _TOK // _NW
_SPW = _TPW // _SEQ
_C = 64
_NCHUNK = _SEQ // _C
_G = 16
_KPS = _C // _G
_NBUF = 4
_U = 4


def _newton_rsqrt(x):
    i = plsc.bitcast(x, jnp.int32)
    i = 0x5F3759DF - (i >> 1)
    y = plsc.bitcast(i, jnp.float32)
    for _ in range(3):
        y = y * (1.5 - 0.5 * x * y * y)
    return y


_mesh = plsc.VectorSubcoreMesh(core_axis_name="c", subcore_axis_name="s")


@functools.partial(
    pl.kernel,
    mesh=_mesh,
    compiler_params=pltpu.CompilerParams(needs_layout_passes=False),
    out_type=jax.ShapeDtypeStruct((_TOK, _HIDDEN), jnp.float32),
    scratch_types=(
        [pltpu.VMEM((_SPW, _NCHUNK, _C), jnp.int32),
         pltpu.VMEM((_C, _HIDDEN), jnp.float32),
         pltpu.VMEM((1, _HIDDEN), jnp.float32),
         pltpu.VMEM((1, _HIDDEN), jnp.float32),
         pltpu.VMEM((_G, _L), jnp.float32),
         pltpu.VMEM((_G, _L), jnp.float32)]
        + [pltpu.VMEM((_G, _HIDDEN), jnp.float32) for _ in range(_NBUF)]
        + [pltpu.SemaphoreType.DMA for _ in range(2 * _NBUF)]
    ),
)
def _emb_kernel(word_hbm, idx_hbm, bias_hbm, gam_hbm, bet_hbm,
                out_hbm, idx_l, bias_v, gam_v, bet_v, acc1_buf, acc2_buf,
                *bufs_sems):
    bufs = bufs_sems[:_NBUF]
    gsem = bufs_sems[_NBUF:2 * _NBUF]
    ssem = bufs_sems[2 * _NBUF:]
    wid = lax.axis_index("s") * _NC + lax.axis_index("c")
    iota = lax.iota(jnp.int32, _L)
    perms = tuple(iota ^ k for k in (8, 4, 2, 1))

    pltpu.sync_copy(idx_hbm.at[wid], idx_l)
    pltpu.sync_copy(gam_hbm, gam_v)
    pltpu.sync_copy(bet_hbm, bet_v)

    def compute_tile(buf, k):
        def p1_row(r, carry):
            zero = jnp.zeros((_L,), jnp.float32)
            a = [zero] * _U
            b = [zero] * _U
            p = k * _G + r
            for jj in range(_NV // _U):
                for u in range(_U):
                    sl = pl.ds((jj * _U + u) * _L, _L)
                    v = buf[r, sl] + bias_v[p, sl]
                    buf[r, sl] = v
                    a[u] = a[u] + v
                    b[u] = b[u] + v * v
            acc1_buf[r, :] = (a[0] + a[1]) + (a[2] + a[3])
            acc2_buf[r, :] = (b[0] + b[1]) + (b[2] + b[3])
            return carry

        lax.fori_loop(0, _G, p1_row, 0)

        def allsum(v):
            for perm in perms:
                v = v + v.at[perm].get(mode="promise_in_bounds")
            return v

        ms, rss = [], []
        for r in range(_G):
            m = allsum(acc1_buf[r, :]) * (1.0 / _HIDDEN)
            var = allsum(acc2_buf[r, :]) * (1.0 / _HIDDEN) - m * m
            ms.append(m)
            rss.append(_newton_rsqrt(var + _EPS))

        def p2_col(j, carry):
            sl = pl.ds(j * _L, _L)
            g = gam_v[0, sl]
            bb = bet_v[0, sl]
            for r in range(_G):
                v = buf[r, sl]
                buf[r, sl] = (v - ms[r]) * rss[r] * g + bb
            return carry

        lax.fori_loop(0, _NV, p2_col, 0)

    for c in range(_NCHUNK):
        pltpu.sync_copy(bias_hbm.at[c], bias_v)

        for k in range(2):
            pltpu.async_copy(
                word_hbm.at[idx_l.at[0, c, pl.ds(k * _G, _G)]],
                bufs[k], gsem[k])

        def seq_body(s, carry, c=c):
            for k in range(_KPS):
                k2 = (k + 2) % _NBUF
                pltpu.make_async_copy(
                    word_hbm.at[pl.ds(0, _G)], bufs[k], gsem[k]).wait()
                compute_tile(bufs[k], k)
                obase = wid * _TPW + s * _SEQ + c * _C + k * _G
                pltpu.async_copy(
                    bufs[k], out_hbm.at[pl.ds(obase, _G)], ssem[k])
                if k < 2:
                    @pl.when(s >= 1)
                    def _():
                        pltpu.make_async_copy(
                            bufs[k2], out_hbm.at[pl.ds(0, _G)],
                            ssem[k2]).wait()
                    pltpu.async_copy(
                        word_hbm.at[idx_l.at[s, c, pl.ds(k2 * _G, _G)]],
                        bufs[k2], gsem[k2])
                else:
                    pltpu.make_async_copy(
                        bufs[k2], out_hbm.at[pl.ds(0, _G)],
                        ssem[k2]).wait()

                    @pl.when(s < _SPW - 1)
                    def _():
                        pltpu.async_copy(
                            word_hbm.at[idx_l.at[s + 1, c,
                                                 pl.ds(k2 * _G, _G)]],
                            bufs[k2], gsem[k2])
            return carry

        lax.fori_loop(0, _SPW, seq_body, 0)

        for k in (2, 3):
            pltpu.make_async_copy(
                bufs[k], out_hbm.at[pl.ds(0, _G)], ssem[k]).wait()


def kernel(input_tokens, word_table, pos_table, type_table, ln_gamma, ln_beta):
    idx = input_tokens.astype(jnp.int32).reshape(_NW, _SPW, _NCHUNK, _C)
    bias = (pos_table[:_SEQ] + type_table[0][None, :]).reshape(
        _NCHUNK, _C, _HIDDEN)
    out = _emb_kernel(word_table, idx, bias,
                      ln_gamma.reshape(1, _HIDDEN),
                      ln_beta.reshape(1, _HIDDEN))
    return out.reshape(_BATCH, _SEQ, _HIDDEN)


# speedup vs baseline: 4.8893x; 1.0031x over previous
import functools

import jax
import jax.numpy as jnp
from jax import lax
from jax.experimental import pallas as pl
from jax.experimental.pallas import tpu as pltpu
from jax.experimental.pallas import tpu_sc as plsc

_VOCAB = 30522
_HIDDEN = 768
_L = 16
_NV = _HIDDEN // _L
_EPS = 1e-12

_NC, _NS = 2, 16
_NW = _NC * _NS
_SEQ = 128
_BATCH = 1024
_TOK = _BATCH * _SEQ
_TPW = _TOK // _NW
_SPW = _TPW // _SEQ
_C = 64
_NCHUNK = _SEQ // _C
_G = 16
_KPS = _C // _G
_NBUF = 4
_U = 4


def _newton_rsqrt(x):
    i = plsc.bitcast(x, jnp.int32)
    i = 0x5F3759DF - (i >> 1)
    y = plsc.bitcast(i, jnp.float32)
    for _ in range(3):
        y = y * (1.5 - 0.5 * x * y * y)
    return y


_mesh = plsc.VectorSubcoreMesh(core_axis_name="c", subcore_axis_name="s")


@functools.partial(
    pl.kernel,
    mesh=_mesh,
    compiler_params=pltpu.CompilerParams(needs_layout_passes=False),
    out_type=jax.ShapeDtypeStruct((_TOK, _HIDDEN), jnp.float32),
    scratch_types=(
        [pltpu.VMEM((_SPW, _NCHUNK, _C), jnp.int32),
         pltpu.VMEM((_C, _HIDDEN), jnp.float32),
         pltpu.VMEM((1, _HIDDEN), jnp.float32),
         pltpu.VMEM((1, _HIDDEN), jnp.float32),
         pltpu.VMEM((_G, _L), jnp.float32),
         pltpu.VMEM((_G, _L), jnp.float32)]
        + [pltpu.VMEM((_G, _HIDDEN), jnp.float32) for _ in range(_NBUF)]
        + [pltpu.SemaphoreType.DMA for _ in range(2 * _NBUF)]
    ),
)
def _emb_kernel(word_hbm, idx_hbm, bias_hbm, gam_hbm, bet_hbm,
                out_hbm, idx_l, bias_v, gam_v, bet_v, acc1_buf, acc2_buf,
                *bufs_sems):
    bufs = bufs_sems[:_NBUF]
    gsem = bufs_sems[_NBUF:2 * _NBUF]
    ssem = bufs_sems[2 * _NBUF:]
    wid = lax.axis_index("s") * _NC + lax.axis_index("c")
    iota = lax.iota(jnp.int32, _L)
    perms = tuple(iota ^ k for k in (8, 4, 2, 1))

    pltpu.sync_copy(idx_hbm.at[wid], idx_l)
    pltpu.sync_copy(gam_hbm, gam_v)
    pltpu.sync_copy(bet_hbm, bet_v)

    def compute_tile(buf, k):
        def p1_row(r, carry):
            zero = jnp.zeros((_L,), jnp.float32)
            a = [zero] * _U
            b = [zero] * _U
            p = k * _G + r
            for jj in range(_NV // _U):
                for u in range(_U):
                    sl = pl.ds((jj * _U + u) * _L, _L)
                    v = buf[r, sl] + bias_v[p, sl]
                    buf[r, sl] = v
                    a[u] = a[u] + v
                    b[u] = b[u] + v * v
            acc1_buf[r, :] = (a[0] + a[1]) + (a[2] + a[3])
            acc2_buf[r, :] = (b[0] + b[1]) + (b[2] + b[3])
            return carry

        lax.fori_loop(0, _G, p1_row, 0)

        def allsum(v):
            for perm in perms:
                v = v + v.at[perm].get(mode="promise_in_bounds")
            return v

        Ss, Ts = [], []
        for r in range(_G):
            m = allsum(acc1_buf[r, :]) * (1.0 / _HIDDEN)
            var = allsum(acc2_buf[r, :]) * (1.0 / _HIDDEN) - m * m
            s = _newton_rsqrt(var + _EPS)
            Ss.append(s)
            Ts.append(-m * s)

        def p2_col(j, carry):
            sl = pl.ds(j * _L, _L)
            g = gam_v[0, sl]
            bb = bet_v[0, sl]
            for r in range(_G):
                v = buf[r, sl]
                buf[r, sl] = (v * Ss[r] + Ts[r]) * g + bb
            return carry

        lax.fori_loop(0, _NV, p2_col, 0)

    for c in range(_NCHUNK):
        pltpu.sync_copy(bias_hbm.at[c], bias_v)

        for k in range(2):
            pltpu.async_copy(
                word_hbm.at[idx_l.at[0, c, pl.ds(k * _G, _G)]],
                bufs[k], gsem[k])

        def seq_body(s, carry, c=c):
            for k in range(_KPS):
                k2 = (k + 2) % _NBUF
                pltpu.make_async_copy(
                    word_hbm.at[pl.ds(0, _G)], bufs[k], gsem[k]).wait()
                compute_tile(bufs[k], k)
                obase = wid * _TPW + s * _SEQ + c * _C + k * _G
                pltpu.async_copy(
                    bufs[k], out_hbm.at[pl.ds(obase, _G)], ssem[k])
                if k < 2:
                    @pl.when(s >= 1)
                    def _():
                        pltpu.make_async_copy(
                            bufs[k2], out_hbm.at[pl.ds(0, _G)],
                            ssem[k2]).wait()
                    pltpu.async_copy(
                        word_hbm.at[idx_l.at[s, c, pl.ds(k2 * _G, _G)]],
                        bufs[k2], gsem[k2])
                else:
                    pltpu.make_async_copy(
                        bufs[k2], out_hbm.at[pl.ds(0, _G)],
                        ssem[k2]).wait()

                    @pl.when(s < _SPW - 1)
                    def _():
                        pltpu.async_copy(
                            word_hbm.at[idx_l.at[s + 1, c,
                                                 pl.ds(k2 * _G, _G)]],
                            bufs[k2], gsem[k2])
            return carry

        lax.fori_loop(0, _SPW, seq_body, 0)

        for k in (2, 3):
            pltpu.make_async_copy(
                bufs[k], out_hbm.at[pl.ds(0, _G)], ssem[k]).wait()


def kernel(input_tokens, word_table, pos_table, type_table, ln_gamma, ln_beta):
    idx = input_tokens.astype(jnp.int32).reshape(_NW, _SPW, _NCHUNK, _C)
    bias = (pos_table[:_SEQ] + type_table[0][None, :]).reshape(
        _NCHUNK, _C, _HIDDEN)
    out = _emb_kernel(word_table, idx, bias,
                      ln_gamma.reshape(1, _HIDDEN),
                      ln_beta.reshape(1, _HIDDEN))
    return out.reshape(_BATCH, _SEQ, _HIDDEN)


# speedup vs baseline: 6.6058x; 1.3511x over previous
import functools

import jax
import jax.numpy as jnp
from jax import lax
from jax.experimental import pallas as pl
from jax.experimental.pallas import tpu as pltpu
from jax.experimental.pallas import tpu_sc as plsc

_VOCAB = 30522
_HIDDEN = 768
_EPS = 1e-12

_NC, _NS = 2, 16
_NW = _NC * _NS
_SEQ = 128
_BATCH = 1024
_TOK = _BATCH * _SEQ
_TPW = _TOK // _NW
_SPW = _TPW // _SEQ
_G = 16
_KPS = _SEQ // _G
_NBUF = 4

_NCH = 4
_TPC = _TOK // _NCH

_BT = 512


_mesh = plsc.VectorSubcoreMesh(core_axis_name="c", subcore_axis_name="s")


@functools.partial(
    pl.kernel,
    mesh=_mesh,
    compiler_params=pltpu.CompilerParams(needs_layout_passes=False),
    out_type=jax.ShapeDtypeStruct((_TOK // _NCH, _HIDDEN), jnp.float32),
    scratch_types=(
        [pltpu.VMEM((_SPW // _NCH, _KPS, _G), jnp.int32)]
        + [pltpu.VMEM((_G, _HIDDEN), jnp.float32) for _ in range(_NBUF)]
        + [pltpu.SemaphoreType.DMA for _ in range(2 * _NBUF)]
    ),
)
def _gather_kernel(word_hbm, idx_hbm, out_hbm, idx_l, *bufs_sems):
    nseq = _SPW // _NCH
    bufs = bufs_sems[:_NBUF]
    gsem = bufs_sems[_NBUF:2 * _NBUF]
    ssem = bufs_sems[2 * _NBUF:]
    wid = lax.axis_index("s") * _NC + lax.axis_index("c")

    pltpu.sync_copy(idx_hbm.at[wid], idx_l)

    for k in range(2):
        pltpu.async_copy(
            word_hbm.at[idx_l.at[0, k]], bufs[k], gsem[k])

    def seq_body(s, carry):
        for k in range(_KPS):
            k2 = (k + 2) % _NBUF
            pltpu.make_async_copy(
                word_hbm.at[pl.ds(0, _G)], bufs[k % _NBUF],
                gsem[k % _NBUF]).wait()
            obase = wid * (_TPW // _NCH) + s * _SEQ + k * _G
            pltpu.async_copy(
                bufs[k % _NBUF], out_hbm.at[pl.ds(obase, _G)],
                ssem[k % _NBUF])
            k2s = (k + 2) % _KPS
            if k < 2:
                @pl.when(s >= 1)
                def _():
                    pltpu.make_async_copy(
                        bufs[k2], out_hbm.at[pl.ds(0, _G)],
                        ssem[k2]).wait()
            else:
                pltpu.make_async_copy(
                    bufs[k2], out_hbm.at[pl.ds(0, _G)], ssem[k2]).wait()
            if k < _KPS - 2:
                pltpu.async_copy(
                    word_hbm.at[idx_l.at[s, k2s]], bufs[k2], gsem[k2])
            else:
                @pl.when(s < nseq - 1)
                def _():
                    pltpu.async_copy(
                        word_hbm.at[idx_l.at[s + 1, k2s]], bufs[k2],
                        gsem[k2])
        return carry

    lax.fori_loop(0, nseq, seq_body, 0)

    for k in (2, 3):
        pltpu.make_async_copy(
            bufs[k], out_hbm.at[pl.ds(0, _G)], ssem[k]).wait()


def _ln_body(x_ref, b_ref, g_ref, be_ref, o_ref):
    x = x_ref[...] + b_ref[...]
    m = jnp.mean(x, axis=1, keepdims=True)
    xc = x - m
    var = jnp.mean(xc * xc, axis=1, keepdims=True)
    y = xc * lax.rsqrt(var + _EPS)
    o_ref[...] = y * g_ref[...] + be_ref[...]


_ln_call = pl.pallas_call(
    _ln_body,
    grid=(_TPC // _BT,),
    in_specs=[
        pl.BlockSpec((_BT, _HIDDEN), lambda i: (i, 0)),
        pl.BlockSpec((_BT, _HIDDEN), lambda i: (0, 0)),
        pl.BlockSpec((1, _HIDDEN), lambda i: (0, 0)),
        pl.BlockSpec((1, _HIDDEN), lambda i: (0, 0)),
    ],
    out_specs=pl.BlockSpec((_BT, _HIDDEN), lambda i: (i, 0)),
    out_shape=jax.ShapeDtypeStruct((_TPC, _HIDDEN), jnp.float32),
)


def kernel(input_tokens, word_table, pos_table, type_table, ln_gamma, ln_beta):
    idx = input_tokens.astype(jnp.int32).reshape(
        _NCH, _NW, _SPW // _NCH, _KPS, _G)
    bias = jnp.tile(pos_table[:_SEQ] + type_table[0][None, :],
                    (_BT // _SEQ, 1))
    gam = ln_gamma.reshape(1, _HIDDEN)
    bet = ln_beta.reshape(1, _HIDDEN)
    outs = []
    for c in range(_NCH):
        g = _gather_kernel(word_table, idx[c])
        outs.append(_ln_call(g, bias, gam, bet))
    out = jnp.concatenate(outs, axis=0)
    return out.reshape(_BATCH, _SEQ, _HIDDEN)
